# Initial kernel scaffold; baseline (speedup 1.0000x reference)
import functools

import jax
import jax.numpy as jnp
from jax import lax
from jax.experimental import pallas as pl
from jax.experimental.pallas import tpu as pltpu
from jax.experimental.pallas import tpu_sc as plsc

NUM_RELS = 38
N_NODES = 100000
N_EDGES = 6400000

NC = 2
NS = 16
L = 16
NW = NC * NS

EPW = N_EDGES // NW
CHUNK = 4000
NCHUNK = EPW // CHUNK
SW_PAD = 512
UNROLL = 8


def _body(src_hbm, dst_hbm, et_hbm, nt_hbm, sw_hbm, out_hbm,
          nt_v, sw_v, src_v, dst_v, et_v, acc_v, sem0, sem1):
  wid = lax.axis_index("s") * NC + lax.axis_index("c")
  base0 = pl.multiple_of(wid * EPW, 8)

  pltpu.sync_copy(nt_hbm, nt_v)
  pltpu.sync_copy(sw_hbm, sw_v)

  def issue(c, slot, sem):
    base = pl.multiple_of(base0 + c * CHUNK, 8)
    pltpu.make_async_copy(src_hbm.at[pl.ds(base, CHUNK)], src_v.at[slot], sem).start()
    pltpu.make_async_copy(dst_hbm.at[pl.ds(base, CHUNK)], dst_v.at[slot], sem).start()
    pltpu.make_async_copy(et_hbm.at[pl.ds(base, CHUNK)], et_v.at[slot], sem).start()

  def drain(slot, sem):
    pltpu.make_async_copy(src_hbm.at[pl.ds(base0, CHUNK)], src_v.at[slot], sem).wait()
    pltpu.make_async_copy(dst_hbm.at[pl.ds(base0, CHUNK)], dst_v.at[slot], sem).wait()
    pltpu.make_async_copy(et_hbm.at[pl.ds(base0, CHUNK)], et_v.at[slot], sem).wait()

  def compute(slot, acc):
    def it(i, acc):
      off = pl.multiple_of(i * (L * UNROLL), L)
      for u in range(UNROLL):
        o = off + u * L
        s = src_v[slot, pl.ds(o, L)]
        d = dst_v[slot, pl.ds(o, L)]
        t = et_v[slot, pl.ds(o, L)]
        ns = plsc.load_gather(nt_v, [s])
        nd = plsc.load_gather(nt_v, [d])
        enc = t * 9 + ns * 3 + nd
        acc = acc + plsc.load_gather(sw_v, [enc])
      return acc
    return lax.fori_loop(0, CHUNK // (L * UNROLL), it, acc)

  issue(0, 0, sem0)
  issue(1, 1, sem1)

  def gloop(g, acc):
    last = g == (NCHUNK // 2 - 1)
    drain(0, sem0)
    @pl.when(jnp.logical_not(last))
    def _():
      issue(2 * g + 2, 0, sem0)
    acc = compute(0, acc)
    drain(1, sem1)
    @pl.when(jnp.logical_not(last))
    def _():
      issue(2 * g + 3, 1, sem1)
    acc = compute(1, acc)
    return acc

  acc = lax.fori_loop(0, NCHUNK // 2, gloop, jnp.zeros((L,), jnp.float32))

  acc_v[...] = acc
  pltpu.sync_copy(acc_v, out_hbm.at[wid])


@jax.jit
def _run(src, dst, et, nt, sw_pad):
  mesh = plsc.VectorSubcoreMesh(core_axis_name="c", subcore_axis_name="s")
  partials = pl.kernel(
      _body,
      out_type=jax.ShapeDtypeStruct((NW, L), jnp.float32),
      mesh=mesh,
      scratch_types=[
          pltpu.VMEM((N_NODES,), jnp.int32),
          pltpu.VMEM((SW_PAD,), jnp.float32),
          pltpu.VMEM((2, CHUNK), jnp.int32),
          pltpu.VMEM((2, CHUNK), jnp.int32),
          pltpu.VMEM((2, CHUNK), jnp.int32),
          pltpu.VMEM((L,), jnp.float32),
          pltpu.SemaphoreType.DMA,
          pltpu.SemaphoreType.DMA,
      ],
  )(src, dst, et, nt, sw_pad)
  return partials.sum()


def kernel(node_type, edge_type, edge_index, text, scorer_weight):
  src = edge_index[0]
  dst = edge_index[1]
  sw = jnp.zeros((SW_PAD,), jnp.float32).at[: NUM_RELS * 9].set(
      scorer_weight.reshape(-1))
  return _run(src, dst, edge_type, node_type, sw)


# speedup vs baseline: 1332.3815x; 1332.3815x over previous
import functools

import jax
import jax.numpy as jnp
from jax import lax
from jax.experimental import pallas as pl
from jax.experimental.pallas import tpu as pltpu
from jax.experimental.pallas import tpu_sc as plsc

NUM_RELS = 38
N_NODES = 100000
N_EDGES = 6400000

NC = 2
NS = 16
L = 16
NW = NC * NS

EPW = N_EDGES // NW
CHUNK = 4000
NCHUNK = EPW // CHUNK
SW_PAD = 512
UNROLL = 8


def _body(src_hbm, dst_hbm, et_hbm, nt_hbm, sw_hbm, out_hbm,
          nt_v, sw_v, src_v0, dst_v0, et_v0, src_v1, dst_v1, et_v1,
          acc_v, sem0, sem1):
  bufs = ((src_v0, dst_v0, et_v0), (src_v1, dst_v1, et_v1))
  wid = lax.axis_index("s") * NC + lax.axis_index("c")
  base0 = pl.multiple_of(wid * EPW, 8)

  pltpu.sync_copy(nt_hbm, nt_v)
  pltpu.sync_copy(sw_hbm, sw_v)

  def issue(c, slot, sem):
    base = pl.multiple_of(base0 + c * CHUNK, 8)
    sb, db, tb = bufs[slot]
    pltpu.make_async_copy(src_hbm.at[pl.ds(base, CHUNK)], sb, sem).start()
    pltpu.make_async_copy(dst_hbm.at[pl.ds(base, CHUNK)], db, sem).start()
    pltpu.make_async_copy(et_hbm.at[pl.ds(base, CHUNK)], tb, sem).start()

  def drain(slot, sem):
    sb, db, tb = bufs[slot]
    pltpu.make_async_copy(src_hbm.at[pl.ds(base0, CHUNK)], sb, sem).wait()
    pltpu.make_async_copy(dst_hbm.at[pl.ds(base0, CHUNK)], db, sem).wait()
    pltpu.make_async_copy(et_hbm.at[pl.ds(base0, CHUNK)], tb, sem).wait()

  def compute(slot, acc):
    sb, db, tb = bufs[slot]
    def it(i, acc):
      off = pl.multiple_of(i * (L * UNROLL), L)
      for u in range(UNROLL):
        o = off + u * L
        s = sb[pl.ds(o, L)]
        d = db[pl.ds(o, L)]
        t = tb[pl.ds(o, L)]
        ns = plsc.load_gather(nt_v, [s])
        nd = plsc.load_gather(nt_v, [d])
        enc = t * 9 + ns * 3 + nd
        acc = acc + plsc.load_gather(sw_v, [enc])
      return acc
    return lax.fori_loop(0, CHUNK // (L * UNROLL), it, acc)

  issue(0, 0, sem0)
  issue(1, 1, sem1)

  def gloop(g, acc):
    last = g == (NCHUNK // 2 - 1)
    drain(0, sem0)
    @pl.when(jnp.logical_not(last))
    def _():
      issue(2 * g + 2, 0, sem0)
    acc = compute(0, acc)
    drain(1, sem1)
    @pl.when(jnp.logical_not(last))
    def _():
      issue(2 * g + 3, 1, sem1)
    acc = compute(1, acc)
    return acc

  acc = lax.fori_loop(0, NCHUNK // 2, gloop, jnp.zeros((L,), jnp.float32))

  acc_v[...] = acc
  pltpu.sync_copy(acc_v, out_hbm.at[wid])


@jax.jit
def _run(src, dst, et, nt, sw_pad):
  mesh = plsc.VectorSubcoreMesh(core_axis_name="c", subcore_axis_name="s")
  partials = pl.kernel(
      _body,
      out_type=jax.ShapeDtypeStruct((NW, L), jnp.float32),
      mesh=mesh,
      compiler_params=pltpu.CompilerParams(needs_layout_passes=False),
      scratch_types=[
          pltpu.VMEM((N_NODES,), jnp.int32),
          pltpu.VMEM((SW_PAD,), jnp.float32),
          pltpu.VMEM((CHUNK,), jnp.int32),
          pltpu.VMEM((CHUNK,), jnp.int32),
          pltpu.VMEM((CHUNK,), jnp.int32),
          pltpu.VMEM((CHUNK,), jnp.int32),
          pltpu.VMEM((CHUNK,), jnp.int32),
          pltpu.VMEM((CHUNK,), jnp.int32),
          pltpu.VMEM((L,), jnp.float32),
          pltpu.SemaphoreType.DMA,
          pltpu.SemaphoreType.DMA,
      ],
  )(src, dst, et, nt, sw_pad)
  return partials.sum()


def kernel(node_type, edge_type, edge_index, text, scorer_weight):
  src = edge_index[0]
  dst = edge_index[1]
  sw = jnp.zeros((SW_PAD,), jnp.float32).at[: NUM_RELS * 9].set(
      scorer_weight.reshape(-1))
  return _run(src, dst, edge_type, node_type, sw)
